# baseline (speedup 1.0000x reference)
import functools

import jax
import jax.numpy as jnp
from jax import lax
from jax.experimental import pallas as pl
from jax.experimental.pallas import tpu as pltpu
from jax.experimental.pallas import tpu_sc as plsc

VOCAB_SIZE = 100000
EMB_DIM = 64
BATCH_SIZE = 1024

_VB = 9216
_RING = 8
_LANE = 128


@functools.lru_cache(maxsize=None)
def _make_sc_gather():
    info = plsc.get_sparse_core_info()
    nc, ns = info.num_cores, info.num_subcores
    nw = nc * ns
    bpw = BATCH_SIZE // nw
    mesh = plsc.VectorSubcoreMesh(core_axis_name="c", subcore_axis_name="s")

    @functools.partial(
        pl.kernel,
        mesh=mesh,
        out_type=jax.ShapeDtypeStruct((BATCH_SIZE, EMB_DIM), jnp.float32),
        scratch_types=[
            pltpu.VMEM((BATCH_SIZE,), jnp.int32),
            pltpu.VMEM((_RING, EMB_DIM, _LANE), jnp.float32),
            pltpu.VMEM((_RING, EMB_DIM), jnp.float32),
            pltpu.SemaphoreType.DMA,
            pltpu.SemaphoreType.DMA,
        ],
        compiler_params=pltpu.CompilerParams(
            use_tc_tiling_on_sc=True, needs_layout_passes=False
        ),
    )
    def sc_gather(tokens_hbm, table_t_hbm, out_hbm, tok_v, ring_v, band_v, sem, osem):
        wid = lax.axis_index("s") * nc + lax.axis_index("c")
        base = wid * bpw
        n_grp = bpw // _RING
        pltpu.sync_copy(tokens_hbm, tok_v)
        lanes = lax.iota(jnp.int32, 16)

        def token_scalar(i):
            c0 = base + (i // 16) * 16
            vec = tok_v[pl.ds(c0, 16)]
            m = lanes == (i % 16)
            return jnp.max(jnp.where(m, vec, 0), axis=0)

        def slab_start(i, b):
            t = token_scalar(i)
            t_al = pl.multiple_of((t // _LANE) * _LANE, _LANE)
            pltpu.async_copy(
                table_t_hbm.at[:, pl.ds(t_al, _LANE)], ring_v.at[b], sem
            )

        def slab_wait(b):
            pltpu.make_async_copy(
                table_t_hbm.at[:, pl.ds(0, _LANE)], ring_v.at[b], sem
            ).wait()

        def out_slice(g):
            off = pl.multiple_of(base + g * _RING, 8)
            return out_hbm.at[pl.ds(off, _RING), :]

        for b in range(_RING):
            slab_start(b, b)

        def group(g, carry):
            @pl.when(g > 0)
            def _():
                pltpu.make_async_copy(band_v, out_slice(0), osem).wait()

            for b in range(_RING):
                slab_wait(b)
                i = g * _RING + b
                t = token_scalar(i)
                col = jnp.full((16,), t % _LANE, jnp.int32)
                slab = ring_v.at[b]
                for jj in range(EMB_DIM // 16):
                    row = lanes + (16 * jj)
                    band_v[b, pl.ds(16 * jj, 16)] = plsc.load_gather(slab, [row, col])

                @pl.when(g < n_grp - 1)
                def _():
                    slab_start(i + _RING, b)

            pltpu.async_copy(band_v, out_slice(g), osem)
            return carry

        lax.fori_loop(0, n_grp, group, 0)
        pltpu.make_async_copy(band_v, out_slice(0), osem).wait()

    return sc_gather


def _proj_body(wt_ref, x_ref, o_ref):
    o_ref[...] = lax.dot_general(
        wt_ref[...].astype(jnp.bfloat16),
        x_ref[...].astype(jnp.bfloat16),
        dimension_numbers=(((0,), (1,)), ((), ())),
        preferred_element_type=jnp.float32,
    ).astype(jnp.bfloat16)


def _proj_t(x, w_t):
    grid = pl.cdiv(VOCAB_SIZE, _VB)
    return pl.pallas_call(
        _proj_body,
        grid=(grid,),
        in_specs=[
            pl.BlockSpec((EMB_DIM, _VB), lambda i: (0, i)),
            pl.BlockSpec((BATCH_SIZE, EMB_DIM), lambda i: (0, 0)),
        ],
        out_specs=pl.BlockSpec((_VB, BATCH_SIZE), lambda i: (i, 0)),
        out_shape=jax.ShapeDtypeStruct((VOCAB_SIZE, BATCH_SIZE), jnp.bfloat16),
        compiler_params=pltpu.CompilerParams(
            dimension_semantics=("parallel",),
        ),
    )(w_t, x)


def kernel(tokens, embed_table, proj_w):
    x = _make_sc_gather()(tokens, jnp.transpose(embed_table))
    out_t = _proj_t(x, jnp.transpose(proj_w))
    return jnp.transpose(out_t)


# speedup vs baseline: 1.0026x; 1.0026x over previous
import functools

import jax
import jax.numpy as jnp
from jax import lax
from jax.experimental import pallas as pl
from jax.experimental.pallas import tpu as pltpu
from jax.experimental.pallas import tpu_sc as plsc

VOCAB_SIZE = 100000
EMB_DIM = 64
BATCH_SIZE = 1024

_VB = 8192
_RING = 8
_LANE = 128


@functools.lru_cache(maxsize=None)
def _make_sc_gather():
    info = plsc.get_sparse_core_info()
    nc, ns = info.num_cores, info.num_subcores
    nw = nc * ns
    bpw = BATCH_SIZE // nw
    mesh = plsc.VectorSubcoreMesh(core_axis_name="c", subcore_axis_name="s")

    @functools.partial(
        pl.kernel,
        mesh=mesh,
        out_type=jax.ShapeDtypeStruct((BATCH_SIZE, EMB_DIM), jnp.float32),
        scratch_types=[
            pltpu.VMEM((BATCH_SIZE,), jnp.int32),
            pltpu.VMEM((_RING, EMB_DIM, _LANE), jnp.float32),
            pltpu.VMEM((_RING, EMB_DIM), jnp.float32),
            pltpu.SemaphoreType.DMA,
            pltpu.SemaphoreType.DMA,
        ],
        compiler_params=pltpu.CompilerParams(
            use_tc_tiling_on_sc=True, needs_layout_passes=False
        ),
    )
    def sc_gather(tokens_hbm, table_t_hbm, out_hbm, tok_v, ring_v, band_v, sem, osem):
        wid = lax.axis_index("s") * nc + lax.axis_index("c")
        base = wid * bpw
        n_grp = bpw // _RING
        pltpu.sync_copy(tokens_hbm, tok_v)
        lanes = lax.iota(jnp.int32, 16)

        def token_scalar(i):
            c0 = base + (i // 16) * 16
            vec = tok_v[pl.ds(c0, 16)]
            m = lanes == (i % 16)
            return jnp.max(jnp.where(m, vec, 0), axis=0)

        def slab_start(i, b):
            t = token_scalar(i)
            t_al = pl.multiple_of((t // _LANE) * _LANE, _LANE)
            pltpu.async_copy(
                table_t_hbm.at[:, pl.ds(t_al, _LANE)], ring_v.at[b], sem
            )

        def slab_wait(b):
            pltpu.make_async_copy(
                table_t_hbm.at[:, pl.ds(0, _LANE)], ring_v.at[b], sem
            ).wait()

        def out_slice(g):
            off = pl.multiple_of(base + g * _RING, 8)
            return out_hbm.at[pl.ds(off, _RING), :]

        for b in range(_RING):
            slab_start(b, b)

        def group(g, carry):
            @pl.when(g > 0)
            def _():
                pltpu.make_async_copy(band_v, out_slice(0), osem).wait()

            for b in range(_RING):
                slab_wait(b)
                i = g * _RING + b
                t = token_scalar(i)
                col = jnp.full((16,), t % _LANE, jnp.int32)
                slab = ring_v.at[b]
                for jj in range(EMB_DIM // 16):
                    row = lanes + (16 * jj)
                    band_v[b, pl.ds(16 * jj, 16)] = plsc.load_gather(slab, [row, col])

                @pl.when(g < n_grp - 1)
                def _():
                    slab_start(i + _RING, b)

            pltpu.async_copy(band_v, out_slice(g), osem)
            return carry

        lax.fori_loop(0, n_grp, group, 0)
        pltpu.make_async_copy(band_v, out_slice(0), osem).wait()

    return sc_gather


def _proj_body(wt_ref, x_ref, o_ref):
    o_ref[...] = lax.dot_general(
        wt_ref[...].astype(jnp.bfloat16),
        x_ref[...].astype(jnp.bfloat16),
        dimension_numbers=(((0,), (1,)), ((), ())),
        preferred_element_type=jnp.float32,
    ).astype(jnp.bfloat16)


def _proj_t(x, w_t):
    grid = pl.cdiv(VOCAB_SIZE, _VB)
    return pl.pallas_call(
        _proj_body,
        grid=(grid,),
        in_specs=[
            pl.BlockSpec((EMB_DIM, _VB), lambda i: (0, i)),
            pl.BlockSpec((BATCH_SIZE, EMB_DIM), lambda i: (0, 0)),
        ],
        out_specs=pl.BlockSpec((_VB, BATCH_SIZE), lambda i: (i, 0)),
        out_shape=jax.ShapeDtypeStruct((VOCAB_SIZE, BATCH_SIZE), jnp.bfloat16),
        compiler_params=pltpu.CompilerParams(
            dimension_semantics=("parallel",),
        ),
    )(w_t, x)


def kernel(tokens, embed_table, proj_w):
    x = _make_sc_gather()(tokens, jnp.transpose(embed_table))
    out_t = _proj_t(x, jnp.transpose(proj_w))
    return jnp.transpose(out_t)
